# baseline (device time: 59418 ns/iter reference)
import os

import jax
import jax.numpy as jnp
from jax import lax
from jax.experimental import pallas as pl
from jax.experimental.pallas import tpu as pltpu

B, H, D, BS = 16, 16, 64, 16
NB = 128
T = NB * BS
Y = 4
NEG = -1e30
_NO_COMM = bool(os.environ.get("KERNEL_NO_COMM"))


def kernel(Q, K, V, bt, lens):
    my_y = lax.axis_index("y")

    j = jnp.arange(NB)
    valid = j[None, :] < lens[:, None]
    local_ids = jnp.arange(NB)[None, None, :] + my_y * NB
    counts = jnp.sum(
        (valid[:, :, None] & (bt[:, :, None] == local_ids)).astype(jnp.int32),
        axis=1,
    )
    tcount = jnp.repeat(counts.astype(jnp.float32), BS, axis=1)

    K2 = jnp.transpose(K.reshape(T, H, D), (1, 2, 0))
    V2 = jnp.transpose(V, (2, 0, 1, 3))
    Q2 = jnp.transpose(Q[:, 0], (1, 0, 2))

    def body(q_ref, k_ref, v_ref, tc_ref, out_ref, comm_ref, send_sems, recv_sems):
        my_x = lax.axis_index("x")
        y = lax.axis_index("y")
        my_z = lax.axis_index("z")

        if not _NO_COMM:
            barrier_sem = pltpu.get_barrier_semaphore()
            for k in range(1, Y):
                pl.semaphore_signal(
                    barrier_sem, inc=1,
                    device_id=(my_x, (y + k) % Y, my_z),
                    device_id_type=pl.DeviceIdType.MESH,
                )
            pl.semaphore_wait(barrier_sem, Y - 1)

        tc = tc_ref[...]

        q = q_ref[...]
        k3 = k_ref[...]
        v3 = v_ref[...].reshape(H, T, D)
        s = lax.dot_general(
            q, k3, (((2,), (1,)), ((0,), (0,))),
            preferred_element_type=jnp.float32,
        ) * (D ** -0.5)
        masked = jnp.where(tc[None] > 0, s, NEG)
        m = jnp.max(masked, axis=-1)
        p = jnp.exp(masked - m[..., None]) * tc[None]
        l = jnp.sum(p, axis=-1)
        o = lax.dot_general(
            p, v3, (((2,), (1,)), ((0,), (0,))),
            preferred_element_type=jnp.float32,
        )
        comm_ref[Y - 1] = jnp.concatenate(
            [o, m[..., None], l[..., None]], axis=-1
        )
        rdmas = []
        if not _NO_COMM:
            for k in range(1, Y):
                rdma = pltpu.make_async_remote_copy(
                    src_ref=comm_ref.at[Y - 1],
                    dst_ref=comm_ref.at[k - 1],
                    send_sem=send_sems.at[k - 1],
                    recv_sem=recv_sems.at[k - 1],
                    device_id=(my_x, (y + k) % Y, my_z),
                    device_id_type=pl.DeviceIdType.MESH,
                )
                rdma.start()
                rdmas.append(rdma)

        mine = comm_ref[Y - 1]
        M = mine[:, :, D]
        L = mine[:, :, D + 1]
        O = mine[:, :, :D]
        for k in range(Y - 1) if not _NO_COMM else []:
            rdmas[k].wait_recv()
            pk = comm_ref[k]
            m2 = pk[:, :, D]
            l2 = pk[:, :, D + 1]
            o2 = pk[:, :, :D]
            Mn = jnp.maximum(M, m2)
            a = jnp.exp(M - Mn)
            c = jnp.exp(m2 - Mn)
            L = L * a + l2 * c
            O = O * a[..., None] + o2 * c[..., None]
            M = Mn
        for r in rdmas:
            r.wait_send()

        res = O / L[..., None]
        out_ref[...] = res.transpose(1, 0, 2)[:, None, :, :]

    return pl.pallas_call(
        body,
        out_shape=jax.ShapeDtypeStruct((B, 1, H, D), jnp.float32),
        in_specs=[pl.BlockSpec(memory_space=pltpu.VMEM)] * 4,
        out_specs=pl.BlockSpec(memory_space=pltpu.VMEM),
        scratch_shapes=[
            pltpu.VMEM((Y, H, B, D + 2), jnp.float32),
            pltpu.SemaphoreType.DMA((Y - 1,)),
            pltpu.SemaphoreType.DMA((Y - 1,)),
        ],
        compiler_params=pltpu.CompilerParams(
            collective_id=None if _NO_COMM else 0
        ),
    )(Q2, K2, V2, tcount)


# device time: 34219 ns/iter; 1.7364x vs baseline; 1.7364x over previous
import os

import jax
import jax.numpy as jnp
from jax import lax
from jax.experimental import pallas as pl
from jax.experimental.pallas import tpu as pltpu

B, H, D, BS = 16, 16, 64, 16
NB = 128
T = NB * BS
Y = 4
NEG = -1e30
_NO_COMM = bool(os.environ.get("KERNEL_NO_COMM"))
_STAGE = os.environ.get("KERNEL_STAGE", "full")


def kernel(Q, K, V, bt, lens):
    my_y = lax.axis_index("y")

    j = jnp.arange(NB)
    valid = j[None, :] < lens[:, None]
    local_ids = jnp.arange(NB)[None, None, :] + my_y * NB
    counts = jnp.sum(
        (valid[:, :, None] & (bt[:, :, None] == local_ids)).astype(jnp.int32),
        axis=1,
    )
    tcount = jnp.repeat(counts.astype(jnp.float32), BS, axis=1)

    K2 = jnp.transpose(K, (2, 0, 1, 3)).astype(jnp.bfloat16)
    V2 = jnp.transpose(V, (2, 0, 1, 3)).astype(jnp.bfloat16)
    Q2 = jnp.transpose(Q[:, 0], (1, 0, 2)).astype(jnp.bfloat16)

    def body(q_ref, k_ref, v_ref, tc_ref, out_ref, comm_ref, send_sems, recv_sems):
        my_x = lax.axis_index("x")
        y = lax.axis_index("y")
        my_z = lax.axis_index("z")

        if not _NO_COMM:
            barrier_sem = pltpu.get_barrier_semaphore()
            for k in range(1, Y):
                pl.semaphore_signal(
                    barrier_sem, inc=1,
                    device_id=(my_x, (y + k) % Y, my_z),
                    device_id_type=pl.DeviceIdType.MESH,
                )
            pl.semaphore_wait(barrier_sem, Y - 1)

        tc = tc_ref[...]

        q = q_ref[...]
        if _NO_COMM and _STAGE == "none":
            out_ref[...] = jnp.broadcast_to(
                q.transpose(1, 0, 2)[:, None, :, :], (B, 1, H, D)
            ).astype(jnp.float32)
            return
        k3 = k_ref[...].reshape(H, T, D)
        v3 = v_ref[...].reshape(H, T, D)
        s = lax.dot_general(
            q, k3, (((2,), (2,)), ((0,), (0,))),
            preferred_element_type=jnp.float32,
        ) * (D ** -0.5)
        if _NO_COMM and _STAGE == "qk":
            mm = jnp.max(s, axis=-1)
            out_ref[...] = jnp.broadcast_to(
                mm.transpose(1, 0)[:, None, :, None], (B, 1, H, D)
            )
            return
        masked = jnp.where(tc[None] > 0, s, NEG)
        m = jnp.max(masked, axis=-1)
        p = jnp.exp(masked - m[..., None]) * tc[None]
        l = jnp.sum(p, axis=-1)
        if _NO_COMM and _STAGE == "sm":
            out_ref[...] = jnp.broadcast_to(
                (l + m).transpose(1, 0)[:, None, :, None], (B, 1, H, D)
            )
            return
        o = lax.dot_general(
            p.astype(jnp.bfloat16), v3, (((2,), (1,)), ((0,), (0,))),
            preferred_element_type=jnp.float32,
        )
        comm_ref[Y - 1] = jnp.concatenate(
            [o, m[..., None], l[..., None]], axis=-1
        )
        rdmas = []
        if not _NO_COMM:
            for k in range(1, Y):
                rdma = pltpu.make_async_remote_copy(
                    src_ref=comm_ref.at[Y - 1],
                    dst_ref=comm_ref.at[k - 1],
                    send_sem=send_sems.at[k - 1],
                    recv_sem=recv_sems.at[k - 1],
                    device_id=(my_x, (y + k) % Y, my_z),
                    device_id_type=pl.DeviceIdType.MESH,
                )
                rdma.start()
                rdmas.append(rdma)

        mine = comm_ref[Y - 1]
        M = mine[:, :, D]
        L = mine[:, :, D + 1]
        O = mine[:, :, :D]
        for k in range(Y - 1) if not _NO_COMM else []:
            rdmas[k].wait_recv()
            pk = comm_ref[k]
            m2 = pk[:, :, D]
            l2 = pk[:, :, D + 1]
            o2 = pk[:, :, :D]
            Mn = jnp.maximum(M, m2)
            a = jnp.exp(M - Mn)
            c = jnp.exp(m2 - Mn)
            L = L * a + l2 * c
            O = O * a[..., None] + o2 * c[..., None]
            M = Mn
        for r in rdmas:
            r.wait_send()

        res = O / L[..., None]
        out_ref[...] = res.transpose(1, 0, 2)[:, None, :, :]

    return pl.pallas_call(
        body,
        out_shape=jax.ShapeDtypeStruct((B, 1, H, D), jnp.float32),
        in_specs=[pl.BlockSpec(memory_space=pltpu.VMEM)] * 4,
        out_specs=pl.BlockSpec(memory_space=pltpu.VMEM),
        scratch_shapes=[
            pltpu.VMEM((Y, H, B, D + 2), jnp.float32),
            pltpu.SemaphoreType.DMA((Y - 1,)),
            pltpu.SemaphoreType.DMA((Y - 1,)),
        ],
        compiler_params=pltpu.CompilerParams(
            collective_id=None if _NO_COMM else 0
        ),
    )(Q2, K2, V2, tcount)
